# Initial kernel scaffold; baseline (speedup 1.0000x reference)
import jax
import jax.numpy as jnp
from jax.experimental import pallas as pl


def kernel(x, edge_index, edge_attr, batch, global_feat, cluster, params):
    raise NotImplementedError("write your pallas kernel here")


# speedup vs baseline: 12.1664x; 12.1664x over previous
import functools

import jax
import jax.numpy as jnp
from jax import lax
from jax.experimental import pallas as pl
from jax.experimental.pallas import tpu as pltpu
from jax.experimental.pallas import tpu_sc as plsc

_N = 10000
_E = 160000
_B = 64
_H = 4
_D = 64
_HD = _H * _D
_F32 = jnp.float32

_NC = 2
_NS = 16
_NW = _NC * _NS
_HALF = _N // _NC
_TAB = 5120
_RPT = _TAB // _NS

_EPW = _E // _NW
_GCH = 39
_GTL = 8
_EPT = _E // _NS
_SCH = 78
_STL = 16

_EBLK = 640
_NEB = _E // _EBLK


def _mm(a, b):
    return jnp.dot(a, b, preferred_element_type=_F32)


def _embed_body(x_ref, wn_ref, bn_ref, wnode_ref, x0_ref, nw_ref):
    x0 = _mm(x_ref[...], wn_ref[...]) + bn_ref[...]
    x0_ref[...] = x0
    nw_ref[...] = _mm(x0, wnode_ref[...])


def _embed_call(x, wn, bn, wnode):
    return pl.pallas_call(
        _embed_body,
        out_shape=(
            jax.ShapeDtypeStruct((_N, _D), _F32),
            jax.ShapeDtypeStruct((_N, _HD), _F32),
        ),
    )(x, wn, bn, wnode)


def _eembed_body(ea_ref, we_ref, be_ref, e_ref):
    t = _mm(ea_ref[...], we_ref[...]) + be_ref[...]
    e_ref[...] = jnp.where(t >= 0, t, 0.2 * t)


def _eembed_call(edge_attr, we, be):
    blk = 2000
    return pl.pallas_call(
        _eembed_body,
        grid=(_E // blk,),
        in_specs=[
            pl.BlockSpec((blk, 41), lambda b: (b, 0)),
            pl.BlockSpec((41, _D), lambda b: (0, 0)),
            pl.BlockSpec((1, _D), lambda b: (0, 0)),
        ],
        out_specs=pl.BlockSpec((blk, _D), lambda b: (b, 0)),
        out_shape=jax.ShapeDtypeStruct((_E, _D), _F32),
    )(edge_attr, we, be)


def _attn_body(gs_ref, gd_ref, e_ref, we_ref, ai_ref, aj_ref,
               al_ref, ps_ref, pq_ref):
    ew = _mm(e_ref[...], we_ref[...])
    xi = jax.nn.softplus(gs_ref[...] + ew)
    xj = jax.nn.softplus(gd_ref[...] + ew)
    al = jax.nn.softplus(_mm(xi, ai_ref[...]) + _mm(xj, aj_ref[...]))
    al_ref[...] = al
    ps_ref[...] = jnp.sum(al, axis=0, keepdims=True)[:, None, :]
    pq_ref[...] = jnp.sum(al * al, axis=0, keepdims=True)[:, None, :]


def _attn_call(gs, gd, e, we, ai, aj):
    return pl.pallas_call(
        _attn_body,
        grid=(_NEB,),
        in_specs=[
            pl.BlockSpec((_EBLK, _HD), lambda b: (b, 0)),
            pl.BlockSpec((_EBLK, _HD), lambda b: (b, 0)),
            pl.BlockSpec((_EBLK, _D), lambda b: (b, 0)),
            pl.BlockSpec((_D, _HD), lambda b: (0, 0)),
            pl.BlockSpec((_HD, 16), lambda b: (0, 0)),
            pl.BlockSpec((_HD, 16), lambda b: (0, 0)),
        ],
        out_specs=(
            pl.BlockSpec((_EBLK, 16), lambda b: (b, 0)),
            pl.BlockSpec((1, 1, 16), lambda b: (b, 0, 0)),
            pl.BlockSpec((1, 1, 16), lambda b: (b, 0, 0)),
        ),
        out_shape=(
            jax.ShapeDtypeStruct((_E, 16), _F32),
            jax.ShapeDtypeStruct((_NEB, 1, 16), _F32),
            jax.ShapeDtypeStruct((_NEB, 1, 16), _F32),
        ),
    )(gs, gd, e, we, ai, aj)


def _weight_body(ps_ref, pq_ref, al_ref, gd_ref, e_ref, we_ref, ex_ref,
                 g_ref, b_ref, wxj_ref, ea_ref):
    mu = jnp.sum(ps_ref[...].reshape(_NEB, 16), axis=0, keepdims=True) / _E
    var = jnp.sum(pq_ref[...].reshape(_NEB, 16), axis=0, keepdims=True) / _E
    var = var - mu * mu
    a = (al_ref[...] - mu) * lax.rsqrt(var + 1e-5) * g_ref[...] + b_ref[...]
    eav = jnp.exp(jax.nn.softplus(a))
    ew = _mm(e_ref[...], we_ref[...])
    xj = jax.nn.softplus(gd_ref[...] + ew)
    wxj_ref[...] = xj * _mm(eav, ex_ref[...])
    ea_ref[...] = eav


def _weight_call(ps, pq, al, gd, e, we, expand, g16, b16):
    return pl.pallas_call(
        _weight_body,
        grid=(_NEB,),
        in_specs=[
            pl.BlockSpec((_NEB, 1, 16), lambda b: (0, 0, 0)),
            pl.BlockSpec((_NEB, 1, 16), lambda b: (0, 0, 0)),
            pl.BlockSpec((_EBLK, 16), lambda b: (b, 0)),
            pl.BlockSpec((_EBLK, _HD), lambda b: (b, 0)),
            pl.BlockSpec((_EBLK, _D), lambda b: (b, 0)),
            pl.BlockSpec((_D, _HD), lambda b: (0, 0)),
            pl.BlockSpec((16, _HD), lambda b: (0, 0)),
            pl.BlockSpec((1, 16), lambda b: (0, 0)),
            pl.BlockSpec((1, 16), lambda b: (0, 0)),
        ],
        out_specs=(
            pl.BlockSpec((_EBLK, _HD), lambda b: (b, 0)),
            pl.BlockSpec((_EBLK, 16), lambda b: (b, 0)),
        ),
        out_shape=(
            jax.ShapeDtypeStruct((_E, _HD), _F32),
            jax.ShapeDtypeStruct((_E, 16), _F32),
        ),
    )(ps, pq, al, gd, e, we, expand, g16, b16)


def _epi_body(num_ref, den_ref, ex_ref, m_ref, bias_ref, g_ref, b_ref,
              wnode_ref, x_ref, nw_ref):
    denb = _mm(den_ref[...], ex_ref[...]) + 1e-16
    ratio = num_ref[...] / denb
    y = _mm(ratio, m_ref[...]) + bias_ref[...]
    mu = jnp.mean(y, axis=0, keepdims=True)
    v = jnp.mean((y - mu) * (y - mu), axis=0, keepdims=True)
    xn = jax.nn.softplus((y - mu) * lax.rsqrt(v + 1e-5) * g_ref[...]
                         + b_ref[...])
    x_ref[...] = xn
    nw_ref[...] = _mm(xn, wnode_ref[...])


def _epi_call(num, den, expand, m, bias, g, b, wnode):
    return pl.pallas_call(
        _epi_body,
        out_shape=(
            jax.ShapeDtypeStruct((_N, _D), _F32),
            jax.ShapeDtypeStruct((_N, _HD), _F32),
        ),
    )(num, den, expand, m, bias, g, b, wnode)


def _pool_body(x_ref, bat_ref, gf_ref, w1a_ref, w1b_ref, b1_ref,
               w2_ref, b2_ref, y_ref):
    xb = x_ref[...]
    oh = (bat_ref[...] == lax.broadcasted_iota(jnp.int32, (1, _B), 1))
    oh = oh.astype(_F32)
    ge = _mm(oh, gf_ref[...])
    h = jax.nn.softplus(_mm(xb, w1a_ref[...]) + _mm(ge, w1b_ref[...])
                        + b1_ref[...])
    s = _mm(h, w2_ref[...]) + b2_ref[...]
    es = jnp.exp(s)
    sums = lax.dot_general(oh, es, (((0,), (0,)), ((), ())),
                           preferred_element_type=_F32)
    den = _mm(oh, sums) + 1e-16
    xw = xb * (es / den)
    y_ref[...] = lax.dot_general(oh, xw, (((0,), (0,)), ((), ())),
                                 preferred_element_type=_F32)


def _pool_call(x, bat, gf, w1a, w1b, b1, w2, b2):
    return pl.pallas_call(
        _pool_body,
        out_shape=jax.ShapeDtypeStruct((_B, _D), _F32),
    )(x, bat, gf, w1a, w1b, b1, w2, b2)


_MESH = plsc.VectorSubcoreMesh(core_axis_name="c", subcore_axis_name="s",
                               num_cores=_NC, num_subcores=_NS)


def _gather_body(nw_ref, src_ref, dst_ref, gs_ref, gd_ref,
                 idx_v, rows_v, idx_t, rows_t, sem):
    c = lax.axis_index("c")
    s = lax.axis_index("s")
    base0 = (s * _NC + c) * _EPW

    def do_range(idx_hbm, out_hbm):
        def step(i, carry):
            bb = base0 + i * 128
            pltpu.sync_copy(idx_hbm.at[pl.ds(bb, 128)], idx_v)
            pltpu.async_copy(nw_ref.at[idx_v], rows_v, sem).wait()
            pltpu.sync_copy(rows_v, out_hbm.at[pl.ds(bb, 128)])
            return carry
        lax.fori_loop(0, _GCH, step, 0)
        bb = base0 + _GCH * 128
        pltpu.sync_copy(idx_hbm.at[pl.ds(bb, _GTL)], idx_t)
        pltpu.async_copy(nw_ref.at[idx_t], rows_t, sem).wait()
        pltpu.sync_copy(rows_t, out_hbm.at[pl.ds(bb, _GTL)])

    do_range(src_ref, gs_ref)
    do_range(dst_ref, gd_ref)


@functools.partial(
    pl.kernel,
    out_type=(
        jax.ShapeDtypeStruct((_E, _HD), _F32),
        jax.ShapeDtypeStruct((_E, _HD), _F32),
    ),
    mesh=_MESH,
    scratch_types=[
        pltpu.VMEM((128,), jnp.int32),
        pltpu.VMEM((128, _HD), _F32),
        pltpu.VMEM((_GTL,), jnp.int32),
        pltpu.VMEM((_GTL, _HD), _F32),
        pltpu.SemaphoreType.DMA,
    ],
)
def _gather2(nw_ref, src_ref, dst_ref, gs_ref, gd_ref,
             idx_v, rows_v, idx_t, rows_t, sem):
    _gather_body(nw_ref, src_ref, dst_ref, gs_ref, gd_ref,
                 idx_v, rows_v, idx_t, rows_t, sem)


def _scatter_chunk(src_ref, wxj_ref, ea_ref, tabn, tabd, nbase,
                   bb, n, vb, eb, ib, jb):
    pltpu.sync_copy(src_ref.at[pl.ds(bb, n)], ib)
    pltpu.sync_copy(wxj_ref.at[pl.ds(bb, n)], vb)
    pltpu.sync_copy(ea_ref.at[pl.ds(bb, n)], eb)
    for j in range(n // 16):
        v = ib[pl.ds(j * 16, 16)]
        vl = v - nbase
        ok = (vl >= 0) & (vl < _HALF)
        jb[pl.ds(j * 16, 16)] = jnp.where(ok, vl, _HALF)
    pltpu.sync_copy(vb, tabn.at[jb], add=True)
    pltpu.sync_copy(eb, tabd.at[jb], add=True)


@functools.partial(
    pl.kernel,
    out_type=(
        jax.ShapeDtypeStruct((_N, _HD), _F32),
        jax.ShapeDtypeStruct((_N, 16), _F32),
    ),
    mesh=_MESH,
    scratch_types=[
        pltpu.VMEM_SHARED((_TAB, _HD), _F32),
        pltpu.VMEM_SHARED((_TAB, 16), _F32),
        pltpu.VMEM((128, _HD), _F32),
        pltpu.VMEM((128, 16), _F32),
        pltpu.VMEM((128,), jnp.int32),
        pltpu.VMEM((128,), jnp.int32),
        pltpu.VMEM((_STL, _HD), _F32),
        pltpu.VMEM((_STL, 16), _F32),
        pltpu.VMEM((_STL,), jnp.int32),
        pltpu.VMEM((_STL,), jnp.int32),
    ],
    compiler_params=pltpu.CompilerParams(use_tc_tiling_on_sc=False),
)
def _scatter(wxj_ref, ea_ref, src_ref, zb_ref, zs_ref, num_ref, den_ref,
             tabn, tabd, vb, eb, ib, jb, vt, et, it_, jt):
    c = lax.axis_index("c")
    s = lax.axis_index("s")
    nbase = c * _HALF
    pltpu.sync_copy(zb_ref.at[pl.ds(s * _RPT, _RPT)],
                    tabn.at[pl.ds(s * _RPT, _RPT)])
    pltpu.sync_copy(zs_ref.at[pl.ds(s * _RPT, _RPT)],
                    tabd.at[pl.ds(s * _RPT, _RPT)])
    plsc.subcore_barrier()

    ebase = s * _EPT

    def step(i, carry):
        _scatter_chunk(src_ref, wxj_ref, ea_ref, tabn, tabd, nbase,
                       ebase + i * 128, 128, vb, eb, ib, jb)
        return carry
    lax.fori_loop(0, _SCH, step, 0)
    _scatter_chunk(src_ref, wxj_ref, ea_ref, tabn, tabd, nbase,
                   ebase + _SCH * 128, _STL, vt, et, it_, jt)
    plsc.subcore_barrier()

    @pl.when(s < _NS - 1)
    def _():
        pltpu.sync_copy(tabn.at[pl.ds(s * _RPT, _RPT)],
                        num_ref.at[pl.ds(nbase + s * _RPT, _RPT)])
        pltpu.sync_copy(tabd.at[pl.ds(s * _RPT, _RPT)],
                        den_ref.at[pl.ds(nbase + s * _RPT, _RPT)])

    @pl.when(s == _NS - 1)
    def _():
        last = _HALF - (_NS - 1) * _RPT
        pltpu.sync_copy(tabn.at[pl.ds((_NS - 1) * _RPT, last)],
                        num_ref.at[pl.ds(nbase + (_NS - 1) * _RPT, last)])
        pltpu.sync_copy(tabd.at[pl.ds((_NS - 1) * _RPT, last)],
                        den_ref.at[pl.ds(nbase + (_NS - 1) * _RPT, last)])


def kernel(x, edge_index, edge_attr, batch, global_feat, cluster, params):
    del cluster
    src = edge_index[0]
    dst = edge_index[1]

    wn, bn_ = params["embed_n"]
    we_emb, be_emb = params["embed_e"]

    eye16 = jnp.eye(16, dtype=_F32)
    expand = jnp.repeat(eye16[:, :_H], _D, axis=1)
    mmean = jnp.tile(jnp.eye(_D, dtype=_F32), (_H, 1)) * (1.0 / _H)

    zeros_big = jnp.zeros((_TAB, _HD), _F32)
    zeros_sm = jnp.zeros((_TAB, 16), _F32)

    e = _eembed_call(edge_attr, we_emb, be_emb.reshape(1, _D))

    layers = params["layers"]
    x_cur, nw = _embed_call(x, wn, bn_.reshape(1, _D),
                            layers[0]["W"][:_D])

    for li, p in enumerate(layers):
        w_edge = p["W"][_D:]
        atti = p["att"][0, :, :_D]
        attj = p["att"][0, :, _D:]
        oh_h = eye16[:_H]
        ai = (atti[:, :, None] * oh_h[:, None, :]).reshape(_HD, 16)
        aj = (attj[:, :, None] * oh_h[:, None, :]).reshape(_HD, 16)
        g16 = jnp.zeros((1, 16), _F32).at[0, :_H].set(p["bn1_g"])
        b16 = jnp.zeros((1, 16), _F32).at[0, :_H].set(p["bn1_b"])

        gs, gd = _gather2(nw, src, dst)
        al, ps, pq = _attn_call(gs, gd, e, w_edge, ai, aj)
        wxj, ea = _weight_call(ps, pq, al, gd, e, w_edge, expand, g16, b16)
        num, den = _scatter(wxj, ea, src, zeros_big, zeros_sm)

        wnode_next = (layers[li + 1]["W"][:_D] if li + 1 < len(layers)
                      else jnp.zeros((_D, _HD), _F32))
        x_cur, nw = _epi_call(num, den, expand, mmean,
                              p["bias"].reshape(1, _D),
                              p["bn_g"].reshape(1, _D),
                              p["bn_b"].reshape(1, _D),
                              wnode_next)

    c = params["comp"]
    y = _pool_call(x_cur, batch.reshape(_N, 1), global_feat,
                   c["W1"][:_D], c["W1"][_D:], c["b1"].reshape(1, 32),
                   c["W2"], c["b2"].reshape(1, 1))
    return y


# speedup vs baseline: 13.2435x; 1.0885x over previous
import functools

import jax
import jax.numpy as jnp
from jax import lax
from jax.experimental import pallas as pl
from jax.experimental.pallas import tpu as pltpu
from jax.experimental.pallas import tpu_sc as plsc

_N = 10000
_E = 160000
_B = 64
_H = 4
_D = 64
_HD = _H * _D
_F32 = jnp.float32

_NC = 2
_NS = 16
_NW = _NC * _NS
_HALF = _N // _NC
_TAB = 5120
_RPT = _TAB // _NS

_EPW = _E // _NW
_GCH = 39
_GTL = 8
_EPT = _E // _NS
_SCH = 78
_STL = 16

_EBLK = 640
_NEB = _E // _EBLK
_DP = 128


def _mm(a, b):
    return jnp.dot(a, b, preferred_element_type=_F32)


def _embed_body(x_ref, wn_ref, bn_ref, x0_ref):
    x0 = _mm(x_ref[...], wn_ref[...]) + bn_ref[...]
    x0_ref[...] = jnp.concatenate(
        [x0, jnp.zeros((_N, _DP - _D), _F32)], axis=1)


def _embed_call(x, wn, bn):
    return pl.pallas_call(
        _embed_body,
        out_shape=jax.ShapeDtypeStruct((_N, _DP), _F32),
    )(x, wn, bn)


def _eembed_body(ea_ref, we_ref, be_ref, e_ref):
    t = _mm(ea_ref[...], we_ref[...]) + be_ref[...]
    e_ref[...] = jnp.where(t >= 0, t, 0.2 * t)


def _eembed_call(edge_attr, we, be):
    blk = 2000
    return pl.pallas_call(
        _eembed_body,
        grid=(_E // blk,),
        in_specs=[
            pl.BlockSpec((blk, 41), lambda b: (b, 0)),
            pl.BlockSpec((41, _D), lambda b: (0, 0)),
            pl.BlockSpec((1, _D), lambda b: (0, 0)),
        ],
        out_specs=pl.BlockSpec((blk, _D), lambda b: (b, 0)),
        out_shape=jax.ShapeDtypeStruct((_E, _D), _F32),
    )(edge_attr, we, be)


_SHIFT = 0.6931472


def _attn_body(xs_ref, xd_ref, e_ref, wn_ref, we_ref, ai_ref, aj_ref,
               al_ref, ps_ref, pq_ref):
    ew = _mm(e_ref[...], we_ref[...])
    xi = jax.nn.softplus(_mm(xs_ref[...], wn_ref[...]) + ew)
    xj = jax.nn.softplus(_mm(xd_ref[...], wn_ref[...]) + ew)
    al = jax.nn.softplus(_mm(xi, ai_ref[...]) + _mm(xj, aj_ref[...]))
    al_ref[...] = al
    d = al - _SHIFT
    ps_ref[...] = jnp.sum(al, axis=0, keepdims=True)[:, None, :]
    pq_ref[...] = jnp.sum(d * d, axis=0, keepdims=True)[:, None, :]


def _attn_call(xs, xd, e, wn, we, ai, aj):
    return pl.pallas_call(
        _attn_body,
        grid=(_NEB,),
        in_specs=[
            pl.BlockSpec((_EBLK, _DP), lambda b: (b, 0)),
            pl.BlockSpec((_EBLK, _DP), lambda b: (b, 0)),
            pl.BlockSpec((_EBLK, _D), lambda b: (b, 0)),
            pl.BlockSpec((_DP, _HD), lambda b: (0, 0)),
            pl.BlockSpec((_D, _HD), lambda b: (0, 0)),
            pl.BlockSpec((_HD, 16), lambda b: (0, 0)),
            pl.BlockSpec((_HD, 16), lambda b: (0, 0)),
        ],
        out_specs=(
            pl.BlockSpec((_EBLK, 16), lambda b: (b, 0)),
            pl.BlockSpec((1, 1, 16), lambda b: (b, 0, 0)),
            pl.BlockSpec((1, 1, 16), lambda b: (b, 0, 0)),
        ),
        out_shape=(
            jax.ShapeDtypeStruct((_E, 16), _F32),
            jax.ShapeDtypeStruct((_NEB, 1, 16), _F32),
            jax.ShapeDtypeStruct((_NEB, 1, 16), _F32),
        ),
    )(xs, xd, e, wn, we, ai, aj)


def _weight_body(ps_ref, pq_ref, al_ref, xd_ref, e_ref, wn_ref, we_ref,
                 ex_ref, g_ref, b_ref, wxj_ref, ea_ref):
    mu = jnp.sum(ps_ref[...].reshape(_NEB, 16), axis=0, keepdims=True) / _E
    q = jnp.sum(pq_ref[...].reshape(_NEB, 16), axis=0, keepdims=True) / _E
    ms = mu - _SHIFT
    var = q - ms * ms
    a = (al_ref[...] - mu) * lax.rsqrt(var + 1e-5) * g_ref[...] + b_ref[...]
    eav = jnp.exp(jax.nn.softplus(a))
    xj = jax.nn.softplus(_mm(xd_ref[...], wn_ref[...])
                         + _mm(e_ref[...], we_ref[...]))
    wxj_ref[...] = xj * _mm(eav, ex_ref[...])
    ea_ref[...] = eav


def _weight_call(ps, pq, al, xd, e, wn, we, expand, g16, b16):
    return pl.pallas_call(
        _weight_body,
        grid=(_NEB,),
        in_specs=[
            pl.BlockSpec((_NEB, 1, 16), lambda b: (0, 0, 0)),
            pl.BlockSpec((_NEB, 1, 16), lambda b: (0, 0, 0)),
            pl.BlockSpec((_EBLK, 16), lambda b: (b, 0)),
            pl.BlockSpec((_EBLK, _DP), lambda b: (b, 0)),
            pl.BlockSpec((_EBLK, _D), lambda b: (b, 0)),
            pl.BlockSpec((_DP, _HD), lambda b: (0, 0)),
            pl.BlockSpec((_D, _HD), lambda b: (0, 0)),
            pl.BlockSpec((16, _HD), lambda b: (0, 0)),
            pl.BlockSpec((1, 16), lambda b: (0, 0)),
            pl.BlockSpec((1, 16), lambda b: (0, 0)),
        ],
        out_specs=(
            pl.BlockSpec((_EBLK, _HD), lambda b: (b, 0)),
            pl.BlockSpec((_EBLK, 16), lambda b: (b, 0)),
        ),
        out_shape=(
            jax.ShapeDtypeStruct((_E, _HD), _F32),
            jax.ShapeDtypeStruct((_E, 16), _F32),
        ),
    )(ps, pq, al, xd, e, wn, we, expand, g16, b16)


def _epi_body(num_ref, den_ref, ex_ref, m_ref, bias_ref, g_ref, b_ref,
              x_ref):
    denb = _mm(den_ref[...], ex_ref[...]) + 1e-16
    ratio = num_ref[...] / denb
    y = _mm(ratio, m_ref[...]) + bias_ref[...]
    mu = jnp.mean(y, axis=0, keepdims=True)
    v = jnp.mean((y - mu) * (y - mu), axis=0, keepdims=True)
    xn = jax.nn.softplus((y - mu) * lax.rsqrt(v + 1e-5) * g_ref[...]
                         + b_ref[...])
    x_ref[...] = jnp.concatenate(
        [xn, jnp.zeros((_N, _DP - _D), _F32)], axis=1)


def _epi_call(num, den, expand, m, bias, g, b):
    return pl.pallas_call(
        _epi_body,
        out_shape=jax.ShapeDtypeStruct((_N, _DP), _F32),
    )(num, den, expand, m, bias, g, b)


def _pool_body(x_ref, bat_ref, gf_ref, w1a_ref, w1b_ref, b1_ref,
               w2_ref, b2_ref, y_ref):
    xb = x_ref[:, :_D]
    oh = (bat_ref[...] == lax.broadcasted_iota(jnp.int32, (1, _B), 1))
    oh = oh.astype(_F32)
    ge = _mm(oh, gf_ref[...])
    h = jax.nn.softplus(_mm(xb, w1a_ref[...]) + _mm(ge, w1b_ref[...])
                        + b1_ref[...])
    s = _mm(h, w2_ref[...]) + b2_ref[...]
    es = jnp.exp(s)
    sums = lax.dot_general(oh, es, (((0,), (0,)), ((), ())),
                           preferred_element_type=_F32)
    den = _mm(oh, sums) + 1e-16
    xw = xb * (es / den)
    y_ref[...] = lax.dot_general(oh, xw, (((0,), (0,)), ((), ())),
                                 preferred_element_type=_F32)


def _pool_call(x, bat, gf, w1a, w1b, b1, w2, b2):
    return pl.pallas_call(
        _pool_body,
        out_shape=jax.ShapeDtypeStruct((_B, _D), _F32),
    )(x, bat, gf, w1a, w1b, b1, w2, b2)


_MESH = plsc.VectorSubcoreMesh(core_axis_name="c", subcore_axis_name="s",
                               num_cores=_NC, num_subcores=_NS)


def _gather_body(nw_ref, src_ref, dst_ref, gs_ref, gd_ref,
                 idx_v, rows_v, idx_t, rows_t, sem):
    c = lax.axis_index("c")
    s = lax.axis_index("s")
    base0 = (s * _NC + c) * _EPW

    def do_range(idx_hbm, out_hbm):
        def step(i, carry):
            bb = base0 + i * 128
            pltpu.sync_copy(idx_hbm.at[pl.ds(bb, 128)], idx_v)
            pltpu.async_copy(nw_ref.at[idx_v], rows_v, sem).wait()
            pltpu.sync_copy(rows_v, out_hbm.at[pl.ds(bb, 128)])
            return carry
        lax.fori_loop(0, _GCH, step, 0)
        bb = base0 + _GCH * 128
        pltpu.sync_copy(idx_hbm.at[pl.ds(bb, _GTL)], idx_t)
        pltpu.async_copy(nw_ref.at[idx_t], rows_t, sem).wait()
        pltpu.sync_copy(rows_t, out_hbm.at[pl.ds(bb, _GTL)])

    do_range(src_ref, gs_ref)
    do_range(dst_ref, gd_ref)


@functools.partial(
    pl.kernel,
    out_type=(
        jax.ShapeDtypeStruct((_E, _DP), _F32),
        jax.ShapeDtypeStruct((_E, _DP), _F32),
    ),
    mesh=_MESH,
    scratch_types=[
        pltpu.VMEM((128,), jnp.int32),
        pltpu.VMEM((128, _DP), _F32),
        pltpu.VMEM((_GTL,), jnp.int32),
        pltpu.VMEM((_GTL, _DP), _F32),
        pltpu.SemaphoreType.DMA,
    ],
)
def _gather2(nw_ref, src_ref, dst_ref, gs_ref, gd_ref,
             idx_v, rows_v, idx_t, rows_t, sem):
    _gather_body(nw_ref, src_ref, dst_ref, gs_ref, gd_ref,
                 idx_v, rows_v, idx_t, rows_t, sem)


def _scatter_chunk(src_ref, wxj_ref, ea_ref, tabn, tabd, nbase,
                   bb, n, vb, eb, ib, jb):
    pltpu.sync_copy(src_ref.at[pl.ds(bb, n)], ib)
    pltpu.sync_copy(wxj_ref.at[pl.ds(bb, n)], vb)
    pltpu.sync_copy(ea_ref.at[pl.ds(bb, n)], eb)
    for j in range(n // 16):
        v = ib[pl.ds(j * 16, 16)]
        vl = v - nbase
        ok = (vl >= 0) & (vl < _HALF)
        jb[pl.ds(j * 16, 16)] = jnp.where(ok, vl, _HALF)
    pltpu.sync_copy(vb, tabn.at[jb], add=True)
    pltpu.sync_copy(eb, tabd.at[jb], add=True)


@functools.partial(
    pl.kernel,
    out_type=(
        jax.ShapeDtypeStruct((_N, _HD), _F32),
        jax.ShapeDtypeStruct((_N, 16), _F32),
    ),
    mesh=_MESH,
    scratch_types=[
        pltpu.VMEM_SHARED((_TAB, _HD), _F32),
        pltpu.VMEM_SHARED((_TAB, 16), _F32),
        pltpu.VMEM((128, _HD), _F32),
        pltpu.VMEM((128, 16), _F32),
        pltpu.VMEM((128,), jnp.int32),
        pltpu.VMEM((128,), jnp.int32),
        pltpu.VMEM((_STL, _HD), _F32),
        pltpu.VMEM((_STL, 16), _F32),
        pltpu.VMEM((_STL,), jnp.int32),
        pltpu.VMEM((_STL,), jnp.int32),
    ],
    compiler_params=pltpu.CompilerParams(use_tc_tiling_on_sc=False),
)
def _scatter(wxj_ref, ea_ref, src_ref, zb_ref, zs_ref, num_ref, den_ref,
             tabn, tabd, vb, eb, ib, jb, vt, et, it_, jt):
    c = lax.axis_index("c")
    s = lax.axis_index("s")
    nbase = c * _HALF
    pltpu.sync_copy(zb_ref.at[pl.ds(s * _RPT, _RPT)],
                    tabn.at[pl.ds(s * _RPT, _RPT)])
    pltpu.sync_copy(zs_ref.at[pl.ds(s * _RPT, _RPT)],
                    tabd.at[pl.ds(s * _RPT, _RPT)])
    plsc.subcore_barrier()

    ebase = s * _EPT

    def step(i, carry):
        _scatter_chunk(src_ref, wxj_ref, ea_ref, tabn, tabd, nbase,
                       ebase + i * 128, 128, vb, eb, ib, jb)
        return carry
    lax.fori_loop(0, _SCH, step, 0)
    _scatter_chunk(src_ref, wxj_ref, ea_ref, tabn, tabd, nbase,
                   ebase + _SCH * 128, _STL, vt, et, it_, jt)
    plsc.subcore_barrier()

    @pl.when(s < _NS - 1)
    def _():
        pltpu.sync_copy(tabn.at[pl.ds(s * _RPT, _RPT)],
                        num_ref.at[pl.ds(nbase + s * _RPT, _RPT)])
        pltpu.sync_copy(tabd.at[pl.ds(s * _RPT, _RPT)],
                        den_ref.at[pl.ds(nbase + s * _RPT, _RPT)])

    @pl.when(s == _NS - 1)
    def _():
        last = _HALF - (_NS - 1) * _RPT
        pltpu.sync_copy(tabn.at[pl.ds((_NS - 1) * _RPT, last)],
                        num_ref.at[pl.ds(nbase + (_NS - 1) * _RPT, last)])
        pltpu.sync_copy(tabd.at[pl.ds((_NS - 1) * _RPT, last)],
                        den_ref.at[pl.ds(nbase + (_NS - 1) * _RPT, last)])


def kernel(x, edge_index, edge_attr, batch, global_feat, cluster, params):
    del cluster
    src = edge_index[0]
    dst = edge_index[1]

    wn, bn_ = params["embed_n"]
    we_emb, be_emb = params["embed_e"]

    eye16 = jnp.eye(16, dtype=_F32)
    expand = jnp.repeat(eye16[:, :_H], _D, axis=1)
    mmean = jnp.tile(jnp.eye(_D, dtype=_F32), (_H, 1)) * (1.0 / _H)

    zeros_big = jnp.zeros((_TAB, _HD), _F32)
    zeros_sm = jnp.zeros((_TAB, 16), _F32)

    e = _eembed_call(edge_attr, we_emb, be_emb.reshape(1, _D))

    layers = params["layers"]
    x_cur = _embed_call(x, wn, bn_.reshape(1, _D))

    for p in layers:
        w_node = jnp.concatenate(
            [p["W"][:_D], jnp.zeros((_DP - _D, _HD), _F32)], axis=0)
        w_edge = p["W"][_D:]
        atti = p["att"][0, :, :_D]
        attj = p["att"][0, :, _D:]
        oh_h = eye16[:_H]
        ai = (atti[:, :, None] * oh_h[:, None, :]).reshape(_HD, 16)
        aj = (attj[:, :, None] * oh_h[:, None, :]).reshape(_HD, 16)
        g16 = jnp.zeros((1, 16), _F32).at[0, :_H].set(p["bn1_g"])
        b16 = jnp.zeros((1, 16), _F32).at[0, :_H].set(p["bn1_b"])

        xs, xd = _gather2(x_cur, src, dst)
        al, ps, pq = _attn_call(xs, xd, e, w_node, w_edge, ai, aj)
        wxj, ea = _weight_call(ps, pq, al, xd, e, w_node, w_edge,
                               expand, g16, b16)
        num, den = _scatter(wxj, ea, src, zeros_big, zeros_sm)
        x_cur = _epi_call(num, den, expand, mmean,
                          p["bias"].reshape(1, _D),
                          p["bn_g"].reshape(1, _D),
                          p["bn_b"].reshape(1, _D))

    c = params["comp"]
    y = _pool_call(x_cur, batch.reshape(_N, 1), global_feat,
                   c["W1"][:_D], c["W1"][_D:], c["b1"].reshape(1, 32),
                   c["W2"], c["b2"].reshape(1, 1))
    return y


# speedup vs baseline: 13.9226x; 1.0513x over previous
import functools

import jax
import jax.numpy as jnp
from jax import lax
from jax.experimental import pallas as pl
from jax.experimental.pallas import tpu as pltpu
from jax.experimental.pallas import tpu_sc as plsc

_N = 10000
_E = 160000
_B = 64
_H = 4
_D = 64
_HD = _H * _D
_F32 = jnp.float32

_NC = 2
_NS = 16
_NW = _NC * _NS
_HALF = _N // _NC
_TAB = 5120
_RPT = _TAB // _NS

_EPW = _E // _NW
_GCH = 39
_GTL = 8
_EPT = _E // _NS
_SCH = 78
_STL = 16

_EBLK = 2000
_NEB = _E // _EBLK
_DP = 128
_W = _HD + 16


def _mm(a, b):
    return jnp.dot(a, b, preferred_element_type=_F32)


def _embed_body(x_ref, wn_ref, bn_ref, x0_ref):
    x0 = _mm(x_ref[...], wn_ref[...]) + bn_ref[...]
    x0_ref[...] = jnp.concatenate(
        [x0, jnp.zeros((_N, _DP - _D), _F32)], axis=1)


def _embed_call(x, wn, bn):
    return pl.pallas_call(
        _embed_body,
        out_shape=jax.ShapeDtypeStruct((_N, _DP), _F32),
    )(x, wn, bn)


def _eembed_body(ea_ref, we_ref, be_ref, e_ref):
    t = _mm(ea_ref[...], we_ref[...]) + be_ref[...]
    e_ref[...] = jnp.where(t >= 0, t, 0.2 * t)


def _eembed_call(edge_attr, we, be):
    blk = 2000
    return pl.pallas_call(
        _eembed_body,
        grid=(_E // blk,),
        in_specs=[
            pl.BlockSpec((blk, 41), lambda b: (b, 0)),
            pl.BlockSpec((41, _D), lambda b: (0, 0)),
            pl.BlockSpec((1, _D), lambda b: (0, 0)),
        ],
        out_specs=pl.BlockSpec((blk, _D), lambda b: (b, 0)),
        out_shape=jax.ShapeDtypeStruct((_E, _D), _F32),
    )(edge_attr, we, be)


_SHIFT = 0.6931472


def _attn_body(xs_ref, xd_ref, e_ref, wn_ref, we_ref, ai_ref, aj_ref,
               al_ref, ps_ref, pq_ref):
    ew = _mm(e_ref[...], we_ref[...])
    xi = jax.nn.softplus(_mm(xs_ref[...], wn_ref[...]) + ew)
    xj = jax.nn.softplus(_mm(xd_ref[...], wn_ref[...]) + ew)
    al = jax.nn.softplus(_mm(xi, ai_ref[...]) + _mm(xj, aj_ref[...]))
    al_ref[...] = al
    d = al - _SHIFT
    ps_ref[...] = jnp.sum(al, axis=0, keepdims=True)[:, None, :]
    pq_ref[...] = jnp.sum(d * d, axis=0, keepdims=True)[:, None, :]


def _attn_call(xs, xd, e, wn, we, ai, aj):
    return pl.pallas_call(
        _attn_body,
        grid=(_NEB,),
        in_specs=[
            pl.BlockSpec((_EBLK, _DP), lambda b: (b, 0)),
            pl.BlockSpec((_EBLK, _DP), lambda b: (b, 0)),
            pl.BlockSpec((_EBLK, _D), lambda b: (b, 0)),
            pl.BlockSpec((_DP, _HD), lambda b: (0, 0)),
            pl.BlockSpec((_D, _HD), lambda b: (0, 0)),
            pl.BlockSpec((_HD, 16), lambda b: (0, 0)),
            pl.BlockSpec((_HD, 16), lambda b: (0, 0)),
        ],
        out_specs=(
            pl.BlockSpec((_EBLK, 16), lambda b: (b, 0)),
            pl.BlockSpec((1, 1, 16), lambda b: (b, 0, 0)),
            pl.BlockSpec((1, 1, 16), lambda b: (b, 0, 0)),
        ),
        out_shape=(
            jax.ShapeDtypeStruct((_E, 16), _F32),
            jax.ShapeDtypeStruct((_NEB, 1, 16), _F32),
            jax.ShapeDtypeStruct((_NEB, 1, 16), _F32),
        ),
    )(xs, xd, e, wn, we, ai, aj)


def _weight_body(ps_ref, pq_ref, al_ref, xd_ref, e_ref, wn_ref, we_ref,
                 ex_ref, g_ref, b_ref, wea_ref):
    mu = jnp.sum(ps_ref[...].reshape(_NEB, 16), axis=0, keepdims=True) / _E
    q = jnp.sum(pq_ref[...].reshape(_NEB, 16), axis=0, keepdims=True) / _E
    ms = mu - _SHIFT
    var = q - ms * ms
    a = (al_ref[...] - mu) * lax.rsqrt(var + 1e-5) * g_ref[...] + b_ref[...]
    eav = jnp.exp(jax.nn.softplus(a))
    xj = jax.nn.softplus(_mm(xd_ref[...], wn_ref[...])
                         + _mm(e_ref[...], we_ref[...]))
    wea_ref[...] = jnp.concatenate([xj * _mm(eav, ex_ref[...]), eav], axis=1)


def _weight_call(ps, pq, al, xd, e, wn, we, expand, g16, b16):
    return pl.pallas_call(
        _weight_body,
        grid=(_NEB,),
        in_specs=[
            pl.BlockSpec((_NEB, 1, 16), lambda b: (0, 0, 0)),
            pl.BlockSpec((_NEB, 1, 16), lambda b: (0, 0, 0)),
            pl.BlockSpec((_EBLK, 16), lambda b: (b, 0)),
            pl.BlockSpec((_EBLK, _DP), lambda b: (b, 0)),
            pl.BlockSpec((_EBLK, _D), lambda b: (b, 0)),
            pl.BlockSpec((_DP, _HD), lambda b: (0, 0)),
            pl.BlockSpec((_D, _HD), lambda b: (0, 0)),
            pl.BlockSpec((16, _HD), lambda b: (0, 0)),
            pl.BlockSpec((1, 16), lambda b: (0, 0)),
            pl.BlockSpec((1, 16), lambda b: (0, 0)),
        ],
        out_specs=pl.BlockSpec((_EBLK, _W), lambda b: (b, 0)),
        out_shape=jax.ShapeDtypeStruct((_E, _W), _F32),
    )(ps, pq, al, xd, e, wn, we, expand, g16, b16)


def _epi_body(nd_ref, ex_ref, m_ref, bias_ref, g_ref, b_ref,
              x_ref):
    denb = _mm(nd_ref[:, _HD:], ex_ref[...]) + 1e-16
    ratio = nd_ref[:, :_HD] / denb
    y = _mm(ratio, m_ref[...]) + bias_ref[...]
    mu = jnp.mean(y, axis=0, keepdims=True)
    v = jnp.mean((y - mu) * (y - mu), axis=0, keepdims=True)
    xn = jax.nn.softplus((y - mu) * lax.rsqrt(v + 1e-5) * g_ref[...]
                         + b_ref[...])
    x_ref[...] = jnp.concatenate(
        [xn, jnp.zeros((_N, _DP - _D), _F32)], axis=1)


def _epi_call(nd, expand, m, bias, g, b):
    return pl.pallas_call(
        _epi_body,
        out_shape=jax.ShapeDtypeStruct((_N, _DP), _F32),
    )(nd, expand, m, bias, g, b)


def _pool_body(x_ref, bat_ref, gf_ref, w1a_ref, w1b_ref, b1_ref,
               w2_ref, b2_ref, y_ref):
    xb = x_ref[:, :_D]
    oh = (bat_ref[...] == lax.broadcasted_iota(jnp.int32, (1, _B), 1))
    oh = oh.astype(_F32)
    ge = _mm(oh, gf_ref[...])
    h = jax.nn.softplus(_mm(xb, w1a_ref[...]) + _mm(ge, w1b_ref[...])
                        + b1_ref[...])
    s = _mm(h, w2_ref[...]) + b2_ref[...]
    es = jnp.exp(s)
    sums = lax.dot_general(oh, es, (((0,), (0,)), ((), ())),
                           preferred_element_type=_F32)
    den = _mm(oh, sums) + 1e-16
    xw = xb * (es / den)
    y_ref[...] = lax.dot_general(oh, xw, (((0,), (0,)), ((), ())),
                                 preferred_element_type=_F32)


def _pool_call(x, bat, gf, w1a, w1b, b1, w2, b2):
    return pl.pallas_call(
        _pool_body,
        out_shape=jax.ShapeDtypeStruct((_B, _D), _F32),
    )(x, bat, gf, w1a, w1b, b1, w2, b2)


_MESH = plsc.VectorSubcoreMesh(core_axis_name="c", subcore_axis_name="s",
                               num_cores=_NC, num_subcores=_NS)


def _gather_body(nw_ref, src_ref, dst_ref, gs_ref, gd_ref,
                 idx_v, rows_v, idx_t, rows_t, sem):
    c = lax.axis_index("c")
    s = lax.axis_index("s")
    base0 = (s * _NC + c) * _EPW

    def do_range(idx_hbm, out_hbm):
        def step(i, carry):
            bb = base0 + i * 128
            pltpu.sync_copy(idx_hbm.at[pl.ds(bb, 128)], idx_v)
            pltpu.async_copy(nw_ref.at[idx_v], rows_v, sem).wait()
            pltpu.sync_copy(rows_v, out_hbm.at[pl.ds(bb, 128)])
            return carry
        lax.fori_loop(0, _GCH, step, 0)
        bb = base0 + _GCH * 128
        pltpu.sync_copy(idx_hbm.at[pl.ds(bb, _GTL)], idx_t)
        pltpu.async_copy(nw_ref.at[idx_t], rows_t, sem).wait()
        pltpu.sync_copy(rows_t, out_hbm.at[pl.ds(bb, _GTL)])

    do_range(src_ref, gs_ref)
    do_range(dst_ref, gd_ref)


@functools.partial(
    pl.kernel,
    out_type=(
        jax.ShapeDtypeStruct((_E, _DP), _F32),
        jax.ShapeDtypeStruct((_E, _DP), _F32),
    ),
    mesh=_MESH,
    scratch_types=[
        pltpu.VMEM((128,), jnp.int32),
        pltpu.VMEM((128, _DP), _F32),
        pltpu.VMEM((_GTL,), jnp.int32),
        pltpu.VMEM((_GTL, _DP), _F32),
        pltpu.SemaphoreType.DMA,
    ],
)
def _gather2(nw_ref, src_ref, dst_ref, gs_ref, gd_ref,
             idx_v, rows_v, idx_t, rows_t, sem):
    _gather_body(nw_ref, src_ref, dst_ref, gs_ref, gd_ref,
                 idx_v, rows_v, idx_t, rows_t, sem)


def _scatter_chunk(src_ref, wea_ref, tab, nbase, bb, n, vb, ib, jb):
    pltpu.sync_copy(src_ref.at[pl.ds(bb, n)], ib)
    pltpu.sync_copy(wea_ref.at[pl.ds(bb, n)], vb)
    for j in range(n // 16):
        v = ib[pl.ds(j * 16, 16)]
        vl = v - nbase
        ok = (vl >= 0) & (vl < _HALF)
        jb[pl.ds(j * 16, 16)] = jnp.where(ok, vl, _HALF)
    pltpu.sync_copy(vb, tab.at[jb], add=True)


@functools.partial(
    pl.kernel,
    out_type=jax.ShapeDtypeStruct((_N, _W), _F32),
    mesh=_MESH,
    scratch_types=[
        pltpu.VMEM_SHARED((_TAB, _W), _F32),
        pltpu.VMEM((128, _W), _F32),
        pltpu.VMEM((128,), jnp.int32),
        pltpu.VMEM((128,), jnp.int32),
        pltpu.VMEM((_STL, _W), _F32),
        pltpu.VMEM((_STL,), jnp.int32),
        pltpu.VMEM((_STL,), jnp.int32),
    ],
    compiler_params=pltpu.CompilerParams(use_tc_tiling_on_sc=False),
)
def _scatter(wea_ref, src_ref, zb_ref, nd_ref,
             tab, vb, ib, jb, vt, it_, jt):
    c = lax.axis_index("c")
    s = lax.axis_index("s")
    nbase = c * _HALF
    pltpu.sync_copy(zb_ref.at[pl.ds(s * _RPT, _RPT)],
                    tab.at[pl.ds(s * _RPT, _RPT)])
    plsc.subcore_barrier()

    ebase = s * _EPT

    def step(i, carry):
        _scatter_chunk(src_ref, wea_ref, tab, nbase,
                       ebase + i * 128, 128, vb, ib, jb)
        return carry
    lax.fori_loop(0, _SCH, step, 0)
    _scatter_chunk(src_ref, wea_ref, tab, nbase,
                   ebase + _SCH * 128, _STL, vt, it_, jt)
    plsc.subcore_barrier()

    @pl.when(s < _NS - 1)
    def _():
        pltpu.sync_copy(tab.at[pl.ds(s * _RPT, _RPT)],
                        nd_ref.at[pl.ds(nbase + s * _RPT, _RPT)])

    @pl.when(s == _NS - 1)
    def _():
        last = _HALF - (_NS - 1) * _RPT
        pltpu.sync_copy(tab.at[pl.ds((_NS - 1) * _RPT, last)],
                        nd_ref.at[pl.ds(nbase + (_NS - 1) * _RPT, last)])


def kernel(x, edge_index, edge_attr, batch, global_feat, cluster, params):
    del cluster
    src = edge_index[0]
    dst = edge_index[1]

    wn, bn_ = params["embed_n"]
    we_emb, be_emb = params["embed_e"]

    eye16 = jnp.eye(16, dtype=_F32)
    expand = jnp.repeat(eye16[:, :_H], _D, axis=1)
    mmean = jnp.tile(jnp.eye(_D, dtype=_F32), (_H, 1)) * (1.0 / _H)

    zeros_big = jnp.zeros((_TAB, _W), _F32)

    e = _eembed_call(edge_attr, we_emb, be_emb.reshape(1, _D))

    layers = params["layers"]
    x_cur = _embed_call(x, wn, bn_.reshape(1, _D))

    for p in layers:
        w_node = jnp.concatenate(
            [p["W"][:_D], jnp.zeros((_DP - _D, _HD), _F32)], axis=0)
        w_edge = p["W"][_D:]
        atti = p["att"][0, :, :_D]
        attj = p["att"][0, :, _D:]
        oh_h = eye16[:_H]
        ai = (atti[:, :, None] * oh_h[:, None, :]).reshape(_HD, 16)
        aj = (attj[:, :, None] * oh_h[:, None, :]).reshape(_HD, 16)
        g16 = jnp.zeros((1, 16), _F32).at[0, :_H].set(p["bn1_g"])
        b16 = jnp.zeros((1, 16), _F32).at[0, :_H].set(p["bn1_b"])

        xs, xd = _gather2(x_cur, src, dst)
        al, ps, pq = _attn_call(xs, xd, e, w_node, w_edge, ai, aj)
        wea = _weight_call(ps, pq, al, xd, e, w_node, w_edge,
                           expand, g16, b16)
        nd = _scatter(wea, src, zeros_big)
        x_cur = _epi_call(nd, expand, mmean,
                          p["bias"].reshape(1, _D),
                          p["bn_g"].reshape(1, _D),
                          p["bn_b"].reshape(1, _D))

    c = params["comp"]
    y = _pool_call(x_cur, batch.reshape(_N, 1), global_feat,
                   c["W1"][:_D], c["W1"][_D:], c["b1"].reshape(1, 32),
                   c["W2"], c["b2"].reshape(1, 1))
    return y


# speedup vs baseline: 14.7227x; 1.0575x over previous
import functools

import jax
import jax.numpy as jnp
from jax import lax
from jax.experimental import pallas as pl
from jax.experimental.pallas import tpu as pltpu
from jax.experimental.pallas import tpu_sc as plsc

_N = 10000
_E = 160000
_B = 64
_H = 4
_D = 64
_HD = _H * _D
_F32 = jnp.float32

_NC = 2
_NS = 16
_NW = _NC * _NS
_HALF = _N // _NC
_TAB = 5120
_RPT = _TAB // _NS

_EPW = _E // _NW
_GCH = 39
_GTL = 8
_EPT = _E // _NS
_SCH = 78
_STL = 16

_EBLK = 2000
_NEB = _E // _EBLK
_DP = 128
_W = _HD + 16


def _mm(a, b):
    return jnp.dot(a, b, preferred_element_type=_F32)


def _embed_body(x_ref, wn_ref, bn_ref, x0_ref):
    x0 = _mm(x_ref[...], wn_ref[...]) + bn_ref[...]
    x0_ref[...] = jnp.concatenate(
        [x0, jnp.zeros((_N, _DP - _D), _F32)], axis=1)


def _embed_call(x, wn, bn):
    return pl.pallas_call(
        _embed_body,
        out_shape=jax.ShapeDtypeStruct((_N, _DP), _F32),
    )(x, wn, bn)


def _eembed_body(ea_ref, we_ref, be_ref, e_ref):
    t = _mm(ea_ref[...], we_ref[...]) + be_ref[...]
    e_ref[...] = jnp.where(t >= 0, t, 0.2 * t)


def _eembed_call(edge_attr, we, be):
    blk = 2000
    return pl.pallas_call(
        _eembed_body,
        grid=(_E // blk,),
        in_specs=[
            pl.BlockSpec((blk, 41), lambda b: (b, 0)),
            pl.BlockSpec((41, _D), lambda b: (0, 0)),
            pl.BlockSpec((1, _D), lambda b: (0, 0)),
        ],
        out_specs=pl.BlockSpec((blk, _D), lambda b: (b, 0)),
        out_shape=jax.ShapeDtypeStruct((_E, _D), _F32),
    )(edge_attr, we, be)


_SHIFT = 0.6931472


def _attn_body(xs_ref, xd_ref, e_ref, wn_ref, we_ref, ai_ref, aj_ref,
               al_ref, ps_ref, pq_ref):
    ew = _mm(e_ref[...], we_ref[...])
    xi = jax.nn.softplus(_mm(xs_ref[...][:, :_D], wn_ref[...]) + ew)
    xj = jax.nn.softplus(_mm(xd_ref[...][:, :_D], wn_ref[...]) + ew)
    al = jax.nn.softplus(_mm(xi, ai_ref[...]) + _mm(xj, aj_ref[...]))
    al_ref[...] = al
    d = al - _SHIFT
    ps_ref[...] = jnp.sum(al, axis=0, keepdims=True)[:, None, :]
    pq_ref[...] = jnp.sum(d * d, axis=0, keepdims=True)[:, None, :]


def _attn_call(xs, xd, e, wn, we, ai, aj):
    return pl.pallas_call(
        _attn_body,
        grid=(_NEB,),
        in_specs=[
            pl.BlockSpec((_EBLK, _DP), lambda b: (b, 0)),
            pl.BlockSpec((_EBLK, _DP), lambda b: (b, 0)),
            pl.BlockSpec((_EBLK, _D), lambda b: (b, 0)),
            pl.BlockSpec((_D, _HD), lambda b: (0, 0)),
            pl.BlockSpec((_D, _HD), lambda b: (0, 0)),
            pl.BlockSpec((_HD, 16), lambda b: (0, 0)),
            pl.BlockSpec((_HD, 16), lambda b: (0, 0)),
        ],
        out_specs=(
            pl.BlockSpec((_EBLK, 16), lambda b: (b, 0)),
            pl.BlockSpec((1, 1, 16), lambda b: (b, 0, 0)),
            pl.BlockSpec((1, 1, 16), lambda b: (b, 0, 0)),
        ),
        out_shape=(
            jax.ShapeDtypeStruct((_E, 16), _F32),
            jax.ShapeDtypeStruct((_NEB, 1, 16), _F32),
            jax.ShapeDtypeStruct((_NEB, 1, 16), _F32),
        ),
    )(xs, xd, e, wn, we, ai, aj)


def _weight_body(ps_ref, pq_ref, al_ref, xd_ref, e_ref, wn_ref, we_ref,
                 ex_ref, g_ref, b_ref, wxj_ref, ea_ref):
    mu = jnp.sum(ps_ref[...].reshape(_NEB, 16), axis=0, keepdims=True) / _E
    q = jnp.sum(pq_ref[...].reshape(_NEB, 16), axis=0, keepdims=True) / _E
    ms = mu - _SHIFT
    var = q - ms * ms
    a = (al_ref[...] - mu) * lax.rsqrt(var + 1e-5) * g_ref[...] + b_ref[...]
    eav = jnp.exp(jax.nn.softplus(a))
    xj = jax.nn.softplus(_mm(xd_ref[...][:, :_D], wn_ref[...])
                         + _mm(e_ref[...], we_ref[...]))
    wxj_ref[...] = xj * _mm(eav, ex_ref[...])
    ea_ref[...] = eav


def _weight_call(ps, pq, al, xd, e, wn, we, expand, g16, b16):
    return pl.pallas_call(
        _weight_body,
        grid=(_NEB,),
        in_specs=[
            pl.BlockSpec((_NEB, 1, 16), lambda b: (0, 0, 0)),
            pl.BlockSpec((_NEB, 1, 16), lambda b: (0, 0, 0)),
            pl.BlockSpec((_EBLK, 16), lambda b: (b, 0)),
            pl.BlockSpec((_EBLK, _DP), lambda b: (b, 0)),
            pl.BlockSpec((_EBLK, _D), lambda b: (b, 0)),
            pl.BlockSpec((_D, _HD), lambda b: (0, 0)),
            pl.BlockSpec((_D, _HD), lambda b: (0, 0)),
            pl.BlockSpec((16, _HD), lambda b: (0, 0)),
            pl.BlockSpec((1, 16), lambda b: (0, 0)),
            pl.BlockSpec((1, 16), lambda b: (0, 0)),
        ],
        out_specs=(
            pl.BlockSpec((_EBLK, _HD), lambda b: (b, 0)),
            pl.BlockSpec((_EBLK, 16), lambda b: (b, 0)),
        ),
        out_shape=(
            jax.ShapeDtypeStruct((_E, _HD), _F32),
            jax.ShapeDtypeStruct((_E, 16), _F32),
        ),
    )(ps, pq, al, xd, e, wn, we, expand, g16, b16)


def _epi_body(num_ref, den_ref, ex_ref, m_ref, bias_ref, g_ref, b_ref,
              x_ref):
    denb = _mm(den_ref[...], ex_ref[...]) + 1e-16
    ratio = num_ref[...] / denb
    y = _mm(ratio, m_ref[...]) + bias_ref[...]
    mu = jnp.mean(y, axis=0, keepdims=True)
    v = jnp.mean((y - mu) * (y - mu), axis=0, keepdims=True)
    xn = jax.nn.softplus((y - mu) * lax.rsqrt(v + 1e-5) * g_ref[...]
                         + b_ref[...])
    x_ref[...] = jnp.concatenate(
        [xn, jnp.zeros((_N, _DP - _D), _F32)], axis=1)


def _epi_call(num, den, expand, m, bias, g, b):
    return pl.pallas_call(
        _epi_body,
        out_shape=jax.ShapeDtypeStruct((_N, _DP), _F32),
    )(num, den, expand, m, bias, g, b)


def _pool_body(x_ref, bat_ref, gf_ref, w1a_ref, w1b_ref, b1_ref,
               w2_ref, b2_ref, y_ref):
    xb = x_ref[:, :_D]
    oh = (bat_ref[...] == lax.broadcasted_iota(jnp.int32, (1, _B), 1))
    oh = oh.astype(_F32)
    ge = _mm(oh, gf_ref[...])
    h = jax.nn.softplus(_mm(xb, w1a_ref[...]) + _mm(ge, w1b_ref[...])
                        + b1_ref[...])
    s = _mm(h, w2_ref[...]) + b2_ref[...]
    es = jnp.exp(s)
    sums = lax.dot_general(oh, es, (((0,), (0,)), ((), ())),
                           preferred_element_type=_F32)
    den = _mm(oh, sums) + 1e-16
    xw = xb * (es / den)
    y_ref[...] = lax.dot_general(oh, xw, (((0,), (0,)), ((), ())),
                                 preferred_element_type=_F32)


def _pool_call(x, bat, gf, w1a, w1b, b1, w2, b2):
    return pl.pallas_call(
        _pool_body,
        out_shape=jax.ShapeDtypeStruct((_B, _D), _F32),
    )(x, bat, gf, w1a, w1b, b1, w2, b2)


_MESH = plsc.VectorSubcoreMesh(core_axis_name="c", subcore_axis_name="s",
                               num_cores=_NC, num_subcores=_NS)


def _gather_body(nw_ref, src_ref, dst_ref, gs_ref, gd_ref,
                 idx_v, rows_v, idx_t, rows_t, sem):
    c = lax.axis_index("c")
    s = lax.axis_index("s")
    base0 = (s * _NC + c) * _EPW

    def do_range(idx_hbm, out_hbm):
        def step(i, carry):
            bb = base0 + i * 128
            pltpu.sync_copy(idx_hbm.at[pl.ds(bb, 128)], idx_v)
            pltpu.async_copy(nw_ref.at[idx_v], rows_v, sem).wait()
            pltpu.sync_copy(rows_v, out_hbm.at[pl.ds(bb, 128)])
            return carry
        lax.fori_loop(0, _GCH, step, 0)
        bb = base0 + _GCH * 128
        pltpu.sync_copy(idx_hbm.at[pl.ds(bb, _GTL)], idx_t)
        pltpu.async_copy(nw_ref.at[idx_t], rows_t, sem).wait()
        pltpu.sync_copy(rows_t, out_hbm.at[pl.ds(bb, _GTL)])

    do_range(src_ref, gs_ref)
    do_range(dst_ref, gd_ref)


@functools.partial(
    pl.kernel,
    out_type=(
        jax.ShapeDtypeStruct((_E, _DP), _F32),
        jax.ShapeDtypeStruct((_E, _DP), _F32),
    ),
    mesh=_MESH,
    scratch_types=[
        pltpu.VMEM((128,), jnp.int32),
        pltpu.VMEM((128, _DP), _F32),
        pltpu.VMEM((_GTL,), jnp.int32),
        pltpu.VMEM((_GTL, _DP), _F32),
        pltpu.SemaphoreType.DMA,
    ],
)
def _gather2(nw_ref, src_ref, dst_ref, gs_ref, gd_ref,
             idx_v, rows_v, idx_t, rows_t, sem):
    _gather_body(nw_ref, src_ref, dst_ref, gs_ref, gd_ref,
                 idx_v, rows_v, idx_t, rows_t, sem)


def _scatter_chunk(src_ref, wxj_ref, ea_ref, tabn, tabd, nbase,
                   bb, n, vb, eb, ib, jb):
    pltpu.sync_copy(src_ref.at[pl.ds(bb, n)], ib)
    pltpu.sync_copy(wxj_ref.at[pl.ds(bb, n)], vb)
    pltpu.sync_copy(ea_ref.at[pl.ds(bb, n)], eb)
    for j in range(n // 16):
        v = ib[pl.ds(j * 16, 16)]
        vl = v - nbase
        ok = (vl >= 0) & (vl < _HALF)
        jb[pl.ds(j * 16, 16)] = jnp.where(ok, vl, _HALF)
    pltpu.sync_copy(vb, tabn.at[jb], add=True)
    pltpu.sync_copy(eb, tabd.at[jb], add=True)


@functools.partial(
    pl.kernel,
    out_type=(
        jax.ShapeDtypeStruct((_N, _HD), _F32),
        jax.ShapeDtypeStruct((_N, 16), _F32),
    ),
    mesh=_MESH,
    scratch_types=[
        pltpu.VMEM_SHARED((_TAB, _HD), _F32),
        pltpu.VMEM_SHARED((_TAB, 16), _F32),
        pltpu.VMEM((128, _HD), _F32),
        pltpu.VMEM((128, 16), _F32),
        pltpu.VMEM((128,), jnp.int32),
        pltpu.VMEM((128,), jnp.int32),
        pltpu.VMEM((_STL, _HD), _F32),
        pltpu.VMEM((_STL, 16), _F32),
        pltpu.VMEM((_STL,), jnp.int32),
        pltpu.VMEM((_STL,), jnp.int32),
    ],
    compiler_params=pltpu.CompilerParams(use_tc_tiling_on_sc=False),
)
def _scatter(wxj_ref, ea_ref, src_ref, zb_ref, zs_ref, num_ref, den_ref,
             tabn, tabd, vb, eb, ib, jb, vt, et, it_, jt):
    c = lax.axis_index("c")
    s = lax.axis_index("s")
    nbase = c * _HALF
    pltpu.sync_copy(zb_ref.at[pl.ds(s * _RPT, _RPT)],
                    tabn.at[pl.ds(s * _RPT, _RPT)])
    pltpu.sync_copy(zs_ref.at[pl.ds(s * _RPT, _RPT)],
                    tabd.at[pl.ds(s * _RPT, _RPT)])
    plsc.subcore_barrier()

    ebase = s * _EPT

    def step(i, carry):
        _scatter_chunk(src_ref, wxj_ref, ea_ref, tabn, tabd, nbase,
                       ebase + i * 128, 128, vb, eb, ib, jb)
        return carry
    lax.fori_loop(0, _SCH, step, 0)
    _scatter_chunk(src_ref, wxj_ref, ea_ref, tabn, tabd, nbase,
                   ebase + _SCH * 128, _STL, vt, et, it_, jt)
    plsc.subcore_barrier()

    @pl.when(s < _NS - 1)
    def _():
        pltpu.sync_copy(tabn.at[pl.ds(s * _RPT, _RPT)],
                        num_ref.at[pl.ds(nbase + s * _RPT, _RPT)])
        pltpu.sync_copy(tabd.at[pl.ds(s * _RPT, _RPT)],
                        den_ref.at[pl.ds(nbase + s * _RPT, _RPT)])

    @pl.when(s == _NS - 1)
    def _():
        last = _HALF - (_NS - 1) * _RPT
        pltpu.sync_copy(tabn.at[pl.ds((_NS - 1) * _RPT, last)],
                        num_ref.at[pl.ds(nbase + (_NS - 1) * _RPT, last)])
        pltpu.sync_copy(tabd.at[pl.ds((_NS - 1) * _RPT, last)],
                        den_ref.at[pl.ds(nbase + (_NS - 1) * _RPT, last)])


def kernel(x, edge_index, edge_attr, batch, global_feat, cluster, params):
    del cluster
    src = edge_index[0]
    dst = edge_index[1]

    wn, bn_ = params["embed_n"]
    we_emb, be_emb = params["embed_e"]

    eye16 = jnp.eye(16, dtype=_F32)
    expand = jnp.repeat(eye16[:, :_H], _D, axis=1)
    mmean = jnp.tile(jnp.eye(_D, dtype=_F32), (_H, 1)) * (1.0 / _H)

    zeros_big = jnp.zeros((_TAB, _HD), _F32)
    zeros_sm = jnp.zeros((_TAB, 16), _F32)

    e = _eembed_call(edge_attr, we_emb, be_emb.reshape(1, _D))

    layers = params["layers"]
    x_cur = _embed_call(x, wn, bn_.reshape(1, _D))

    for p in layers:
        w_node = p["W"][:_D]
        w_edge = p["W"][_D:]
        atti = p["att"][0, :, :_D]
        attj = p["att"][0, :, _D:]
        oh_h = eye16[:_H]
        ai = (atti[:, :, None] * oh_h[:, None, :]).reshape(_HD, 16)
        aj = (attj[:, :, None] * oh_h[:, None, :]).reshape(_HD, 16)
        g16 = jnp.zeros((1, 16), _F32).at[0, :_H].set(p["bn1_g"])
        b16 = jnp.zeros((1, 16), _F32).at[0, :_H].set(p["bn1_b"])

        xs, xd = _gather2(x_cur, src, dst)
        al, ps, pq = _attn_call(xs, xd, e, w_node, w_edge, ai, aj)
        wxj, ea = _weight_call(ps, pq, al, xd, e, w_node, w_edge,
                               expand, g16, b16)
        num, den = _scatter(wxj, ea, src, zeros_big, zeros_sm)
        x_cur = _epi_call(num, den, expand, mmean,
                          p["bias"].reshape(1, _D),
                          p["bn_g"].reshape(1, _D),
                          p["bn_b"].reshape(1, _D))

    c = params["comp"]
    y = _pool_call(x_cur, batch.reshape(_N, 1), global_feat,
                   c["W1"][:_D], c["W1"][_D:], c["b1"].reshape(1, 32),
                   c["W2"], c["b2"].reshape(1, 1))
    return y


# speedup vs baseline: 15.0010x; 1.0189x over previous
import functools

import jax
import jax.numpy as jnp
from jax import lax
from jax.experimental import pallas as pl
from jax.experimental.pallas import tpu as pltpu
from jax.experimental.pallas import tpu_sc as plsc

_N = 10000
_E = 160000
_B = 64
_H = 4
_D = 64
_HD = _H * _D
_F32 = jnp.float32

_NC = 2
_NS = 16
_NW = _NC * _NS
_HALF = _N // _NC
_TAB = 5120
_RPT = _TAB // _NS

_EPW = _E // _NW
_GCH = 39
_GTL = 8
_EPT = _E // _NS
_SCH = 78
_STL = 16

_EBLK = 4000
_NEB = _E // _EBLK
_DP = 128
_W = _HD + 16


def _mm(a, b):
    return jnp.dot(a, b, preferred_element_type=_F32)


def _embed_body(x_ref, wn_ref, bn_ref, x0_ref):
    x0 = _mm(x_ref[...], wn_ref[...]) + bn_ref[...]
    x0_ref[...] = jnp.concatenate(
        [x0, jnp.zeros((_N, _DP - _D), _F32)], axis=1)


def _embed_call(x, wn, bn):
    return pl.pallas_call(
        _embed_body,
        out_shape=jax.ShapeDtypeStruct((_N, _DP), _F32),
    )(x, wn, bn)


def _eembed_body(ea_ref, we_ref, be_ref, e_ref):
    t = _mm(ea_ref[...], we_ref[...]) + be_ref[...]
    e_ref[...] = jnp.where(t >= 0, t, 0.2 * t)


def _eembed_call(edge_attr, we, be):
    blk = 2000
    return pl.pallas_call(
        _eembed_body,
        grid=(_E // blk,),
        in_specs=[
            pl.BlockSpec((blk, 41), lambda b: (b, 0)),
            pl.BlockSpec((41, _D), lambda b: (0, 0)),
            pl.BlockSpec((1, _D), lambda b: (0, 0)),
        ],
        out_specs=pl.BlockSpec((blk, _D), lambda b: (b, 0)),
        out_shape=jax.ShapeDtypeStruct((_E, _D), _F32),
    )(edge_attr, we, be)


_SHIFT = 0.6931472


def _attn_body(xs_ref, xd_ref, e_ref, wn_ref, we_ref, ai_ref, aj_ref,
               al_ref, ps_ref, pq_ref):
    ew = _mm(e_ref[...], we_ref[...])
    xi = jax.nn.softplus(_mm(xs_ref[...][:, :_D], wn_ref[...]) + ew)
    xj = jax.nn.softplus(_mm(xd_ref[...][:, :_D], wn_ref[...]) + ew)
    al = jax.nn.softplus(_mm(xi, ai_ref[...]) + _mm(xj, aj_ref[...]))
    al_ref[...] = al
    d = al - _SHIFT
    ps_ref[...] = jnp.sum(al, axis=0, keepdims=True)[:, None, :]
    pq_ref[...] = jnp.sum(d * d, axis=0, keepdims=True)[:, None, :]


def _attn_call(xs, xd, e, wn, we, ai, aj):
    return pl.pallas_call(
        _attn_body,
        grid=(_NEB,),
        in_specs=[
            pl.BlockSpec((_EBLK, _DP), lambda b: (b, 0)),
            pl.BlockSpec((_EBLK, _DP), lambda b: (b, 0)),
            pl.BlockSpec((_EBLK, _D), lambda b: (b, 0)),
            pl.BlockSpec((_D, _HD), lambda b: (0, 0)),
            pl.BlockSpec((_D, _HD), lambda b: (0, 0)),
            pl.BlockSpec((_HD, 16), lambda b: (0, 0)),
            pl.BlockSpec((_HD, 16), lambda b: (0, 0)),
        ],
        out_specs=(
            pl.BlockSpec((_EBLK, 16), lambda b: (b, 0)),
            pl.BlockSpec((1, 1, 16), lambda b: (b, 0, 0)),
            pl.BlockSpec((1, 1, 16), lambda b: (b, 0, 0)),
        ),
        out_shape=(
            jax.ShapeDtypeStruct((_E, 16), _F32),
            jax.ShapeDtypeStruct((_NEB, 1, 16), _F32),
            jax.ShapeDtypeStruct((_NEB, 1, 16), _F32),
        ),
    )(xs, xd, e, wn, we, ai, aj)


def _weight_body(ps_ref, pq_ref, al_ref, xd_ref, e_ref, wn_ref, we_ref,
                 ex_ref, g_ref, b_ref, wxj_ref, ea_ref):
    mu = jnp.sum(ps_ref[...].reshape(_NEB, 16), axis=0, keepdims=True) / _E
    q = jnp.sum(pq_ref[...].reshape(_NEB, 16), axis=0, keepdims=True) / _E
    ms = mu - _SHIFT
    var = q - ms * ms
    a = (al_ref[...] - mu) * lax.rsqrt(var + 1e-5) * g_ref[...] + b_ref[...]
    eav = jnp.exp(jax.nn.softplus(a))
    xj = jax.nn.softplus(_mm(xd_ref[...][:, :_D], wn_ref[...])
                         + _mm(e_ref[...], we_ref[...]))
    wxj_ref[...] = xj * _mm(eav, ex_ref[...])
    ea_ref[...] = eav


def _weight_call(ps, pq, al, xd, e, wn, we, expand, g16, b16):
    return pl.pallas_call(
        _weight_body,
        grid=(_NEB,),
        in_specs=[
            pl.BlockSpec((_NEB, 1, 16), lambda b: (0, 0, 0)),
            pl.BlockSpec((_NEB, 1, 16), lambda b: (0, 0, 0)),
            pl.BlockSpec((_EBLK, 16), lambda b: (b, 0)),
            pl.BlockSpec((_EBLK, _DP), lambda b: (b, 0)),
            pl.BlockSpec((_EBLK, _D), lambda b: (b, 0)),
            pl.BlockSpec((_D, _HD), lambda b: (0, 0)),
            pl.BlockSpec((_D, _HD), lambda b: (0, 0)),
            pl.BlockSpec((16, _HD), lambda b: (0, 0)),
            pl.BlockSpec((1, 16), lambda b: (0, 0)),
            pl.BlockSpec((1, 16), lambda b: (0, 0)),
        ],
        out_specs=(
            pl.BlockSpec((_EBLK, _HD), lambda b: (b, 0)),
            pl.BlockSpec((_EBLK, 16), lambda b: (b, 0)),
        ),
        out_shape=(
            jax.ShapeDtypeStruct((_E, _HD), _F32),
            jax.ShapeDtypeStruct((_E, 16), _F32),
        ),
    )(ps, pq, al, xd, e, wn, we, expand, g16, b16)


def _epi_body(num_ref, den_ref, ex_ref, m_ref, bias_ref, g_ref, b_ref,
              x_ref):
    denb = _mm(den_ref[...], ex_ref[...]) + 1e-16
    ratio = num_ref[...] / denb
    y = _mm(ratio, m_ref[...]) + bias_ref[...]
    mu = jnp.mean(y, axis=0, keepdims=True)
    v = jnp.mean((y - mu) * (y - mu), axis=0, keepdims=True)
    xn = jax.nn.softplus((y - mu) * lax.rsqrt(v + 1e-5) * g_ref[...]
                         + b_ref[...])
    x_ref[...] = jnp.concatenate(
        [xn, jnp.zeros((_N, _DP - _D), _F32)], axis=1)


def _epi_call(num, den, expand, m, bias, g, b):
    return pl.pallas_call(
        _epi_body,
        out_shape=jax.ShapeDtypeStruct((_N, _DP), _F32),
    )(num, den, expand, m, bias, g, b)


def _pool_body(x_ref, bat_ref, gf_ref, w1a_ref, w1b_ref, b1_ref,
               w2_ref, b2_ref, y_ref):
    xb = x_ref[:, :_D]
    oh = (bat_ref[...] == lax.broadcasted_iota(jnp.int32, (1, _B), 1))
    oh = oh.astype(_F32)
    ge = _mm(oh, gf_ref[...])
    h = jax.nn.softplus(_mm(xb, w1a_ref[...]) + _mm(ge, w1b_ref[...])
                        + b1_ref[...])
    s = _mm(h, w2_ref[...]) + b2_ref[...]
    es = jnp.exp(s)
    sums = lax.dot_general(oh, es, (((0,), (0,)), ((), ())),
                           preferred_element_type=_F32)
    den = _mm(oh, sums) + 1e-16
    xw = xb * (es / den)
    y_ref[...] = lax.dot_general(oh, xw, (((0,), (0,)), ((), ())),
                                 preferred_element_type=_F32)


def _pool_call(x, bat, gf, w1a, w1b, b1, w2, b2):
    return pl.pallas_call(
        _pool_body,
        out_shape=jax.ShapeDtypeStruct((_B, _D), _F32),
    )(x, bat, gf, w1a, w1b, b1, w2, b2)


_MESH = plsc.VectorSubcoreMesh(core_axis_name="c", subcore_axis_name="s",
                               num_cores=_NC, num_subcores=_NS)


def _gather_body(nw_ref, src_ref, dst_ref, gs_ref, gd_ref,
                 idx_v, rows_v, idx_t, rows_t, sem):
    c = lax.axis_index("c")
    s = lax.axis_index("s")
    base0 = (s * _NC + c) * _EPW

    def do_range(idx_hbm, out_hbm):
        def step(i, carry):
            bb = base0 + i * 128
            pltpu.sync_copy(idx_hbm.at[pl.ds(bb, 128)], idx_v)
            pltpu.async_copy(nw_ref.at[idx_v], rows_v, sem).wait()
            pltpu.sync_copy(rows_v, out_hbm.at[pl.ds(bb, 128)])
            return carry
        lax.fori_loop(0, _GCH, step, 0)
        bb = base0 + _GCH * 128
        pltpu.sync_copy(idx_hbm.at[pl.ds(bb, _GTL)], idx_t)
        pltpu.async_copy(nw_ref.at[idx_t], rows_t, sem).wait()
        pltpu.sync_copy(rows_t, out_hbm.at[pl.ds(bb, _GTL)])

    do_range(src_ref, gs_ref)
    do_range(dst_ref, gd_ref)


@functools.partial(
    pl.kernel,
    out_type=(
        jax.ShapeDtypeStruct((_E, _DP), _F32),
        jax.ShapeDtypeStruct((_E, _DP), _F32),
    ),
    mesh=_MESH,
    scratch_types=[
        pltpu.VMEM((128,), jnp.int32),
        pltpu.VMEM((128, _DP), _F32),
        pltpu.VMEM((_GTL,), jnp.int32),
        pltpu.VMEM((_GTL, _DP), _F32),
        pltpu.SemaphoreType.DMA,
    ],
)
def _gather2(nw_ref, src_ref, dst_ref, gs_ref, gd_ref,
             idx_v, rows_v, idx_t, rows_t, sem):
    _gather_body(nw_ref, src_ref, dst_ref, gs_ref, gd_ref,
                 idx_v, rows_v, idx_t, rows_t, sem)


def _scatter_chunk(src_ref, wxj_ref, ea_ref, tabn, tabd, nbase,
                   bb, n, vb, eb, ib, jb):
    pltpu.sync_copy(src_ref.at[pl.ds(bb, n)], ib)
    pltpu.sync_copy(wxj_ref.at[pl.ds(bb, n)], vb)
    pltpu.sync_copy(ea_ref.at[pl.ds(bb, n)], eb)
    for j in range(n // 16):
        v = ib[pl.ds(j * 16, 16)]
        vl = v - nbase
        ok = (vl >= 0) & (vl < _HALF)
        jb[pl.ds(j * 16, 16)] = jnp.where(ok, vl, _HALF)
    pltpu.sync_copy(vb, tabn.at[jb], add=True)
    pltpu.sync_copy(eb, tabd.at[jb], add=True)


@functools.partial(
    pl.kernel,
    out_type=(
        jax.ShapeDtypeStruct((_N, _HD), _F32),
        jax.ShapeDtypeStruct((_N, 16), _F32),
    ),
    mesh=_MESH,
    scratch_types=[
        pltpu.VMEM_SHARED((_TAB, _HD), _F32),
        pltpu.VMEM_SHARED((_TAB, 16), _F32),
        pltpu.VMEM((128, _HD), _F32),
        pltpu.VMEM((128, 16), _F32),
        pltpu.VMEM((128,), jnp.int32),
        pltpu.VMEM((128,), jnp.int32),
        pltpu.VMEM((_STL, _HD), _F32),
        pltpu.VMEM((_STL, 16), _F32),
        pltpu.VMEM((_STL,), jnp.int32),
        pltpu.VMEM((_STL,), jnp.int32),
    ],
    compiler_params=pltpu.CompilerParams(use_tc_tiling_on_sc=False),
)
def _scatter(wxj_ref, ea_ref, src_ref, zb_ref, zs_ref, num_ref, den_ref,
             tabn, tabd, vb, eb, ib, jb, vt, et, it_, jt):
    c = lax.axis_index("c")
    s = lax.axis_index("s")
    nbase = c * _HALF
    pltpu.sync_copy(zb_ref.at[pl.ds(s * _RPT, _RPT)],
                    tabn.at[pl.ds(s * _RPT, _RPT)])
    pltpu.sync_copy(zs_ref.at[pl.ds(s * _RPT, _RPT)],
                    tabd.at[pl.ds(s * _RPT, _RPT)])
    plsc.subcore_barrier()

    ebase = s * _EPT

    def step(i, carry):
        _scatter_chunk(src_ref, wxj_ref, ea_ref, tabn, tabd, nbase,
                       ebase + i * 128, 128, vb, eb, ib, jb)
        return carry
    lax.fori_loop(0, _SCH, step, 0)
    _scatter_chunk(src_ref, wxj_ref, ea_ref, tabn, tabd, nbase,
                   ebase + _SCH * 128, _STL, vt, et, it_, jt)
    plsc.subcore_barrier()

    @pl.when(s < _NS - 1)
    def _():
        pltpu.sync_copy(tabn.at[pl.ds(s * _RPT, _RPT)],
                        num_ref.at[pl.ds(nbase + s * _RPT, _RPT)])
        pltpu.sync_copy(tabd.at[pl.ds(s * _RPT, _RPT)],
                        den_ref.at[pl.ds(nbase + s * _RPT, _RPT)])

    @pl.when(s == _NS - 1)
    def _():
        last = _HALF - (_NS - 1) * _RPT
        pltpu.sync_copy(tabn.at[pl.ds((_NS - 1) * _RPT, last)],
                        num_ref.at[pl.ds(nbase + (_NS - 1) * _RPT, last)])
        pltpu.sync_copy(tabd.at[pl.ds((_NS - 1) * _RPT, last)],
                        den_ref.at[pl.ds(nbase + (_NS - 1) * _RPT, last)])


def kernel(x, edge_index, edge_attr, batch, global_feat, cluster, params):
    del cluster
    src = edge_index[0]
    dst = edge_index[1]

    wn, bn_ = params["embed_n"]
    we_emb, be_emb = params["embed_e"]

    eye16 = jnp.eye(16, dtype=_F32)
    expand = jnp.repeat(eye16[:, :_H], _D, axis=1)
    mmean = jnp.tile(jnp.eye(_D, dtype=_F32), (_H, 1)) * (1.0 / _H)

    zeros_big = jnp.zeros((_TAB, _HD), _F32)
    zeros_sm = jnp.zeros((_TAB, 16), _F32)

    e = _eembed_call(edge_attr, we_emb, be_emb.reshape(1, _D))

    layers = params["layers"]
    x_cur = _embed_call(x, wn, bn_.reshape(1, _D))

    for p in layers:
        w_node = p["W"][:_D]
        w_edge = p["W"][_D:]
        atti = p["att"][0, :, :_D]
        attj = p["att"][0, :, _D:]
        oh_h = eye16[:_H]
        ai = (atti[:, :, None] * oh_h[:, None, :]).reshape(_HD, 16)
        aj = (attj[:, :, None] * oh_h[:, None, :]).reshape(_HD, 16)
        g16 = jnp.zeros((1, 16), _F32).at[0, :_H].set(p["bn1_g"])
        b16 = jnp.zeros((1, 16), _F32).at[0, :_H].set(p["bn1_b"])

        xs, xd = _gather2(x_cur, src, dst)
        al, ps, pq = _attn_call(xs, xd, e, w_node, w_edge, ai, aj)
        wxj, ea = _weight_call(ps, pq, al, xd, e, w_node, w_edge,
                               expand, g16, b16)
        num, den = _scatter(wxj, ea, src, zeros_big, zeros_sm)
        x_cur = _epi_call(num, den, expand, mmean,
                          p["bias"].reshape(1, _D),
                          p["bn_g"].reshape(1, _D),
                          p["bn_b"].reshape(1, _D))

    c = params["comp"]
    y = _pool_call(x_cur, batch.reshape(_N, 1), global_feat,
                   c["W1"][:_D], c["W1"][_D:], c["b1"].reshape(1, 32),
                   c["W2"], c["b2"].reshape(1, 1))
    return y


# speedup vs baseline: 15.8064x; 1.0537x over previous
import functools

import jax
import jax.numpy as jnp
from jax import lax
from jax.experimental import pallas as pl
from jax.experimental.pallas import tpu as pltpu
from jax.experimental.pallas import tpu_sc as plsc

_N = 10000
_E = 160000
_B = 64
_H = 4
_D = 64
_HD = _H * _D
_F32 = jnp.float32

_NC = 2
_NS = 16
_NW = _NC * _NS
_HALF = _N // _NC
_TAB = 5120
_RPT = _TAB // _NS

_EH = _E // 2
_NCH = _EH // 128
_GFC = _NCH // _NW
_GXW = _NCH - _GFC * _NW
_SFC = _NCH // _NS
_SXW = _NCH - _SFC * _NS

_EBLK = 4000
_NEBH = _EH // _EBLK
_DP = 128
_W = _HD + 16


def _mm(a, b):
    return jnp.dot(a, b, preferred_element_type=_F32)


def _embed_body(x_ref, wn_ref, bn_ref, x0_ref):
    x0 = _mm(x_ref[...], wn_ref[...]) + bn_ref[...]
    x0_ref[...] = jnp.concatenate(
        [x0, jnp.zeros((_N, _DP - _D), _F32)], axis=1)


def _embed_call(x, wn, bn):
    return pl.pallas_call(
        _embed_body,
        out_shape=jax.ShapeDtypeStruct((_N, _DP), _F32),
    )(x, wn, bn)


def _eembed_body(ea_ref, we_ref, be_ref, e_ref):
    t = _mm(ea_ref[...], we_ref[...]) + be_ref[...]
    e_ref[...] = jnp.where(t >= 0, t, 0.2 * t)


def _eembed_call(edge_attr, we, be):
    blk = 2000
    return pl.pallas_call(
        _eembed_body,
        grid=(_E // blk,),
        in_specs=[
            pl.BlockSpec((blk, 41), lambda b: (b, 0)),
            pl.BlockSpec((41, _D), lambda b: (0, 0)),
            pl.BlockSpec((1, _D), lambda b: (0, 0)),
        ],
        out_specs=pl.BlockSpec((blk, _D), lambda b: (b, 0)),
        out_shape=jax.ShapeDtypeStruct((_E, _D), _F32),
    )(edge_attr, we, be)


_SHIFT = 0.6931472


def _attn_body(xs_ref, xd_ref, e_ref, wn_ref, we_ref, ai_ref, aj_ref,
               al_ref, ps_ref, pq_ref):
    ew = _mm(e_ref[...], we_ref[...])
    xi = jax.nn.softplus(_mm(xs_ref[...][:, :_D], wn_ref[...]) + ew)
    xj = jax.nn.softplus(_mm(xd_ref[...][:, :_D], wn_ref[...]) + ew)
    al = jax.nn.softplus(_mm(xi, ai_ref[...]) + _mm(xj, aj_ref[...]))
    al_ref[...] = al
    d = al - _SHIFT
    ps_ref[...] = jnp.sum(al, axis=0, keepdims=True)[:, None, :]
    pq_ref[...] = jnp.sum(d * d, axis=0, keepdims=True)[:, None, :]


def _attn_call(h, xs, xd, e, wn, we, ai, aj):
    return pl.pallas_call(
        _attn_body,
        grid=(_NEBH,),
        in_specs=[
            pl.BlockSpec((_EBLK, _DP), lambda b: (b, 0)),
            pl.BlockSpec((_EBLK, _DP), lambda b: (b, 0)),
            pl.BlockSpec((_EBLK, _D), lambda b, h=h: (h * _NEBH + b, 0)),
            pl.BlockSpec((_D, _HD), lambda b: (0, 0)),
            pl.BlockSpec((_D, _HD), lambda b: (0, 0)),
            pl.BlockSpec((_HD, 16), lambda b: (0, 0)),
            pl.BlockSpec((_HD, 16), lambda b: (0, 0)),
        ],
        out_specs=(
            pl.BlockSpec((_EBLK, 16), lambda b: (b, 0)),
            pl.BlockSpec((1, 1, 16), lambda b: (b, 0, 0)),
            pl.BlockSpec((1, 1, 16), lambda b: (b, 0, 0)),
        ),
        out_shape=(
            jax.ShapeDtypeStruct((_EH, 16), _F32),
            jax.ShapeDtypeStruct((_NEBH, 1, 16), _F32),
            jax.ShapeDtypeStruct((_NEBH, 1, 16), _F32),
        ),
    )(xs, xd, e, wn, we, ai, aj)


def _weight_body(ps0_ref, pq0_ref, ps1_ref, pq1_ref, al_ref, xd_ref,
                 e_ref, wn_ref, we_ref, ex_ref, g_ref, b_ref,
                 wxj_ref, ea_ref):
    ps = (jnp.sum(ps0_ref[...].reshape(_NEBH, 16), axis=0, keepdims=True)
          + jnp.sum(ps1_ref[...].reshape(_NEBH, 16), axis=0, keepdims=True))
    pq = (jnp.sum(pq0_ref[...].reshape(_NEBH, 16), axis=0, keepdims=True)
          + jnp.sum(pq1_ref[...].reshape(_NEBH, 16), axis=0, keepdims=True))
    mu = ps / _E
    q = pq / _E
    ms = mu - _SHIFT
    var = q - ms * ms
    a = (al_ref[...] - mu) * lax.rsqrt(var + 1e-5) * g_ref[...] + b_ref[...]
    eav = jnp.exp(jax.nn.softplus(a))
    xj = jax.nn.softplus(_mm(xd_ref[...][:, :_D], wn_ref[...])
                         + _mm(e_ref[...], we_ref[...]))
    wxj_ref[...] = xj * _mm(eav, ex_ref[...])
    ea_ref[...] = eav


def _weight_call(h, ps0, pq0, ps1, pq1, al, xd, e, wn, we,
                 expand, g16, b16):
    return pl.pallas_call(
        _weight_body,
        grid=(_NEBH,),
        in_specs=[
            pl.BlockSpec((_NEBH, 1, 16), lambda b: (0, 0, 0)),
            pl.BlockSpec((_NEBH, 1, 16), lambda b: (0, 0, 0)),
            pl.BlockSpec((_NEBH, 1, 16), lambda b: (0, 0, 0)),
            pl.BlockSpec((_NEBH, 1, 16), lambda b: (0, 0, 0)),
            pl.BlockSpec((_EBLK, 16), lambda b: (b, 0)),
            pl.BlockSpec((_EBLK, _DP), lambda b: (b, 0)),
            pl.BlockSpec((_EBLK, _D), lambda b, h=h: (h * _NEBH + b, 0)),
            pl.BlockSpec((_D, _HD), lambda b: (0, 0)),
            pl.BlockSpec((_D, _HD), lambda b: (0, 0)),
            pl.BlockSpec((16, _HD), lambda b: (0, 0)),
            pl.BlockSpec((1, 16), lambda b: (0, 0)),
            pl.BlockSpec((1, 16), lambda b: (0, 0)),
        ],
        out_specs=(
            pl.BlockSpec((_EBLK, _HD), lambda b: (b, 0)),
            pl.BlockSpec((_EBLK, 16), lambda b: (b, 0)),
        ),
        out_shape=(
            jax.ShapeDtypeStruct((_EH, _HD), _F32),
            jax.ShapeDtypeStruct((_EH, 16), _F32),
        ),
    )(ps0, pq0, ps1, pq1, al, xd, e, wn, we, expand, g16, b16)


def _epi_body(num0_ref, num1_ref, den0_ref, den1_ref, ex_ref, m_ref,
              bias_ref, g_ref, b_ref, x_ref):
    den = den0_ref[...] + den1_ref[...]
    denb = _mm(den, ex_ref[...]) + 1e-16
    ratio = (num0_ref[...] + num1_ref[...]) / denb
    y = _mm(ratio, m_ref[...]) + bias_ref[...]
    mu = jnp.mean(y, axis=0, keepdims=True)
    v = jnp.mean((y - mu) * (y - mu), axis=0, keepdims=True)
    xn = jax.nn.softplus((y - mu) * lax.rsqrt(v + 1e-5) * g_ref[...]
                         + b_ref[...])
    x_ref[...] = jnp.concatenate(
        [xn, jnp.zeros((_N, _DP - _D), _F32)], axis=1)


def _epi_call(num0, num1, den0, den1, expand, m, bias, g, b):
    return pl.pallas_call(
        _epi_body,
        out_shape=jax.ShapeDtypeStruct((_N, _DP), _F32),
    )(num0, num1, den0, den1, expand, m, bias, g, b)


def _pool_body(x_ref, bat_ref, gf_ref, w1a_ref, w1b_ref, b1_ref,
               w2_ref, b2_ref, y_ref):
    xb = x_ref[:, :_D]
    oh = (bat_ref[...] == lax.broadcasted_iota(jnp.int32, (1, _B), 1))
    oh = oh.astype(_F32)
    ge = _mm(oh, gf_ref[...])
    h = jax.nn.softplus(_mm(xb, w1a_ref[...]) + _mm(ge, w1b_ref[...])
                        + b1_ref[...])
    s = _mm(h, w2_ref[...]) + b2_ref[...]
    es = jnp.exp(s)
    sums = lax.dot_general(oh, es, (((0,), (0,)), ((), ())),
                           preferred_element_type=_F32)
    den = _mm(oh, sums) + 1e-16
    xw = xb * (es / den)
    y_ref[...] = lax.dot_general(oh, xw, (((0,), (0,)), ((), ())),
                                 preferred_element_type=_F32)


def _pool_call(x, bat, gf, w1a, w1b, b1, w2, b2):
    return pl.pallas_call(
        _pool_body,
        out_shape=jax.ShapeDtypeStruct((_B, _D), _F32),
    )(x, bat, gf, w1a, w1b, b1, w2, b2)


_MESH = plsc.VectorSubcoreMesh(core_axis_name="c", subcore_axis_name="s",
                               num_cores=_NC, num_subcores=_NS)


@functools.partial(
    pl.kernel,
    out_type=(
        jax.ShapeDtypeStruct((_EH, _DP), _F32),
        jax.ShapeDtypeStruct((_EH, _DP), _F32),
    ),
    mesh=_MESH,
    scratch_types=[
        pltpu.VMEM((128,), jnp.int32),
        pltpu.VMEM((128, _DP), _F32),
        pltpu.SemaphoreType.DMA,
    ],
)
def _gather2(tab_ref, src_ref, dst_ref, gs_ref, gd_ref,
             idx_v, rows_v, sem):
    c = lax.axis_index("c")
    s = lax.axis_index("s")
    w = s * _NC + c

    def do_range(idx_hbm, out_hbm):
        def chunk(k):
            bb = k * 128
            pltpu.sync_copy(idx_hbm.at[pl.ds(bb, 128)], idx_v)
            pltpu.async_copy(tab_ref.at[idx_v], rows_v, sem).wait()
            pltpu.sync_copy(rows_v, out_hbm.at[pl.ds(bb, 128)])

        def step(i, carry):
            chunk(w + i * _NW)
            return carry
        lax.fori_loop(0, _GFC, step, 0)

        @pl.when(w < _GXW)
        def _():
            chunk(w + _GFC * _NW)

    do_range(src_ref, gs_ref)
    do_range(dst_ref, gd_ref)


@functools.partial(
    pl.kernel,
    out_type=(
        jax.ShapeDtypeStruct((_N, _HD), _F32),
        jax.ShapeDtypeStruct((_N, 16), _F32),
    ),
    mesh=_MESH,
    scratch_types=[
        pltpu.VMEM_SHARED((_TAB, _HD), _F32),
        pltpu.VMEM_SHARED((_TAB, 16), _F32),
        pltpu.VMEM((128, _HD), _F32),
        pltpu.VMEM((128, 16), _F32),
        pltpu.VMEM((128,), jnp.int32),
        pltpu.VMEM((128,), jnp.int32),
    ],
    compiler_params=pltpu.CompilerParams(use_tc_tiling_on_sc=False),
)
def _scatter(wxj_ref, ea_ref, src_ref, zb_ref, zs_ref, num_ref, den_ref,
             tabn, tabd, vb, eb, ib, jb):
    c = lax.axis_index("c")
    s = lax.axis_index("s")
    nbase = c * _HALF
    pltpu.sync_copy(zb_ref.at[pl.ds(s * _RPT, _RPT)],
                    tabn.at[pl.ds(s * _RPT, _RPT)])
    pltpu.sync_copy(zs_ref.at[pl.ds(s * _RPT, _RPT)],
                    tabd.at[pl.ds(s * _RPT, _RPT)])
    plsc.subcore_barrier()

    def chunk(k):
        bb = k * 128
        pltpu.sync_copy(src_ref.at[pl.ds(bb, 128)], ib)
        pltpu.sync_copy(wxj_ref.at[pl.ds(bb, 128)], vb)
        pltpu.sync_copy(ea_ref.at[pl.ds(bb, 128)], eb)
        for j in range(8):
            v = ib[pl.ds(j * 16, 16)]
            vl = v - nbase
            ok = (vl >= 0) & (vl < _HALF)
            jb[pl.ds(j * 16, 16)] = jnp.where(ok, vl, _HALF)
        pltpu.sync_copy(vb, tabn.at[jb], add=True)
        pltpu.sync_copy(eb, tabd.at[jb], add=True)

    def step(i, carry):
        chunk(s + i * _NS)
        return carry
    lax.fori_loop(0, _SFC, step, 0)

    @pl.when(s < _SXW)
    def _():
        chunk(s + _SFC * _NS)
    plsc.subcore_barrier()

    @pl.when(s < _NS - 1)
    def _():
        pltpu.sync_copy(tabn.at[pl.ds(s * _RPT, _RPT)],
                        num_ref.at[pl.ds(nbase + s * _RPT, _RPT)])
        pltpu.sync_copy(tabd.at[pl.ds(s * _RPT, _RPT)],
                        den_ref.at[pl.ds(nbase + s * _RPT, _RPT)])

    @pl.when(s == _NS - 1)
    def _():
        last = _HALF - (_NS - 1) * _RPT
        pltpu.sync_copy(tabn.at[pl.ds((_NS - 1) * _RPT, last)],
                        num_ref.at[pl.ds(nbase + (_NS - 1) * _RPT, last)])
        pltpu.sync_copy(tabd.at[pl.ds((_NS - 1) * _RPT, last)],
                        den_ref.at[pl.ds(nbase + (_NS - 1) * _RPT, last)])


def kernel(x, edge_index, edge_attr, batch, global_feat, cluster, params):
    del cluster
    src = edge_index[0]
    dst = edge_index[1]
    src0, src1 = src[:_EH], src[_EH:]
    dst0, dst1 = dst[:_EH], dst[_EH:]

    wn, bn_ = params["embed_n"]
    we_emb, be_emb = params["embed_e"]

    eye16 = jnp.eye(16, dtype=_F32)
    expand = jnp.repeat(eye16[:, :_H], _D, axis=1)
    mmean = jnp.tile(jnp.eye(_D, dtype=_F32), (_H, 1)) * (1.0 / _H)

    zeros_big = jnp.zeros((_TAB, _HD), _F32)
    zeros_sm = jnp.zeros((_TAB, 16), _F32)

    e = _eembed_call(edge_attr, we_emb, be_emb.reshape(1, _D))

    layers = params["layers"]
    x_cur = _embed_call(x, wn, bn_.reshape(1, _D))

    for p in layers:
        w_node = p["W"][:_D]
        w_edge = p["W"][_D:]
        atti = p["att"][0, :, :_D]
        attj = p["att"][0, :, _D:]
        oh_h = eye16[:_H]
        ai = (atti[:, :, None] * oh_h[:, None, :]).reshape(_HD, 16)
        aj = (attj[:, :, None] * oh_h[:, None, :]).reshape(_HD, 16)
        g16 = jnp.zeros((1, 16), _F32).at[0, :_H].set(p["bn1_g"])
        b16 = jnp.zeros((1, 16), _F32).at[0, :_H].set(p["bn1_b"])

        xs0, xd0 = _gather2(x_cur, src0, dst0)
        xs1, xd1 = _gather2(x_cur, src1, dst1)
        al0, ps0, pq0 = _attn_call(0, xs0, xd0, e, w_node, w_edge, ai, aj)
        al1, ps1, pq1 = _attn_call(1, xs1, xd1, e, w_node, w_edge, ai, aj)
        wxj0, ea0 = _weight_call(0, ps0, pq0, ps1, pq1, al0, xd0, e,
                                 w_node, w_edge, expand, g16, b16)
        num0, den0 = _scatter(wxj0, ea0, src0, zeros_big, zeros_sm)
        wxj1, ea1 = _weight_call(1, ps0, pq0, ps1, pq1, al1, xd1, e,
                                 w_node, w_edge, expand, g16, b16)
        num1, den1 = _scatter(wxj1, ea1, src1, zeros_big, zeros_sm)
        x_cur = _epi_call(num0, num1, den0, den1, expand, mmean,
                          p["bias"].reshape(1, _D),
                          p["bn_g"].reshape(1, _D),
                          p["bn_b"].reshape(1, _D))

    c = params["comp"]
    y = _pool_call(x_cur, batch.reshape(_N, 1), global_feat,
                   c["W1"][:_D], c["W1"][_D:], c["b1"].reshape(1, 32),
                   c["W2"], c["b2"].reshape(1, 1))
    return y
